# baseline (speedup 1.0000x reference)
import jax
import jax.numpy as jnp
from jax import lax
from jax.experimental import pallas as pl
from jax.experimental.pallas import tpu as pltpu

NUM_HEADS = 4
M = 64
NUM_SHIFT = 3
B, IN_DIM, STATE, OUT_DIM, N_ADDR = 128, 256, 512, 256, 2048
EPS = 1e-8
PER_HEAD = 2 * M + NUM_SHIFT + 1 + (M + 2)
UPDATE_SIZE = NUM_HEADS * PER_HEAD
READ_SIZE = NUM_HEADS * M
CTRL_IN = IN_DIM + STATE + READ_SIZE

BB = 8
GRID = B // BB


def _sigmoid(x):
    return 1.0 / (1.0 + jnp.exp(-x))


def _softplus(x):
    return jnp.maximum(x, 0.0) + jnp.log(1.0 + jnp.exp(-jnp.abs(x)))


def _dot(a, b):
    return jnp.dot(a, b, preferred_element_type=jnp.float32)


def _dot_t_rhs(a, b):
    return lax.dot_general(a, b, (((1,), (1,)), ((), ())),
                           preferred_element_type=jnp.float32)


def _dot_t_lhs(a, b):
    return lax.dot_general(a, b, (((0,), (0,)), ((), ())),
                           preferred_element_type=jnp.float32)


def _ntm_kernel(x_ref, s_ref, wt_ref, mem_hbm, wtd_ref,
                W_f_ref, b_f_ref, W_h_ref, b_h_ref,
                W_s_ref, b_s_ref, W_o_ref, b_o_ref, W_u_ref, b_u_ref,
                out_ref, sn_ref, wtn_ref, memn_hbm, wtdyn_ref,
                wmix_ref, mem_buf, in_sems, out_sems):
    i = pl.program_id(0)

    def in_copy(step, slot):
        return pltpu.make_async_copy(
            mem_hbm.at[pl.ds(step * BB, BB)],
            mem_buf.at[slot],
            in_sems.at[slot],
        )

    def out_copy(step, slot):
        return pltpu.make_async_copy(
            mem_buf.at[slot],
            memn_hbm.at[pl.ds(step * BB, BB)],
            out_sems.at[slot],
        )

    @pl.when(i == 0)
    def _():
        in_copy(0, 0).start()

    @pl.when(i >= 1)
    def _():
        out_copy(i - 1, (i + 1) % 2).wait()

    @pl.when(i + 1 < GRID)
    def _():
        in_copy(i + 1, (i + 1) % 2).start()

    slot = i % 2
    in_copy(i, slot).wait()
    mem_ref = mem_buf.at[slot]

    x = x_ref[...].astype(jnp.bfloat16)
    s = s_ref[...].astype(jnp.bfloat16)

    f = _sigmoid(_dot(s, W_f_ref[0:STATE, :]) + _dot(x, W_f_ref[STATE:, :])
                 + b_f_ref[...])
    h = _sigmoid(_dot(s, W_h_ref[0:STATE, :]) + _dot(x, W_h_ref[STATE:, :])
                 + b_h_ref[...])

    hh = lax.broadcasted_iota(jnp.int32, (NUM_HEADS, N_ADDR), 0)
    nn = lax.broadcasted_iota(jnp.int32, (NUM_HEADS, N_ADDR), 1)
    delta = jnp.where((hh == 0) & (nn == 0), 1.0, 0.0)

    reads = []
    for b in range(BB):
        wt_b = wt_ref[b]
        wtd_b = wtd_ref[b]
        f0 = f[b:b + 1, 0:1]
        f1 = f[b:b + 1, 1:2]
        h0 = h[b:b + 1, 0:1]
        wt_dyn_b = (1.0 - h0) * wt_b + h0 * wtd_b
        wtdyn_ref[b] = wt_dyn_b
        wmix_b = ((1.0 - f0) * (1.0 - f1) * delta
                  + (1.0 - f0) * f1 * wt_b
                  + f0 * (1.0 - f1) * wt_dyn_b)
        wmix_ref[b] = wmix_b
        reads.append(_dot_t_rhs(wmix_b, mem_ref[b]))

    read_h = [jnp.concatenate([reads[b][hd:hd + 1, :] for b in range(BB)],
                              axis=0) for hd in range(NUM_HEADS)]

    read_bf = [r.astype(jnp.bfloat16) for r in read_h]

    def ctrl_mm(W_ref, bias_ref):
        z = (_dot(x, W_ref[0:IN_DIM, :])
             + _dot(s, W_ref[IN_DIM:IN_DIM + STATE, :]))
        for hd in range(NUM_HEADS):
            r0 = IN_DIM + STATE + hd * M
            z = z + _dot(read_bf[hd], W_ref[r0:r0 + M, :])
        return z + bias_ref[...]

    sn_ref[...] = _sigmoid(ctrl_mm(W_s_ref, b_s_ref))
    out_ref[...] = _sigmoid(ctrl_mm(W_o_ref, b_o_ref))
    u = ctrl_mm(W_u_ref, b_u_ref)

    erase_h, add_h, key_h, sh_h, gam_h, bet_h, g_h = [], [], [], [], [], [], []
    for hd in range(NUM_HEADS):
        o = hd * PER_HEAD
        erase_h.append(_sigmoid(u[:, o:o + M]))
        add_h.append(u[:, o + M:o + 2 * M])
        raw_sh = u[:, o + 2 * M:o + 2 * M + NUM_SHIFT]
        raw_sh = raw_sh - jnp.max(raw_sh, axis=-1, keepdims=True)
        e = jnp.exp(raw_sh)
        sh_h.append(e / jnp.sum(e, axis=-1, keepdims=True))
        gam_h.append(1.0 + _softplus(
            u[:, o + 2 * M + NUM_SHIFT:o + 2 * M + NUM_SHIFT + 1]))
        co = o + 2 * M + NUM_SHIFT + 1
        key_h.append(u[:, co:co + M])
        bet_h.append(_softplus(u[:, co + M:co + M + 1]))
        g_h.append(_sigmoid(u[:, co + M + 1:co + M + 2]))

    ones_row = jnp.full((1, M), 1.0, dtype=jnp.float32)

    for b in range(BB):
        mem_b = mem_ref[b]
        key_b = jnp.concatenate([key_h[hd][b:b + 1, :]
                                 for hd in range(NUM_HEADS)], axis=0)
        add_b = jnp.concatenate([add_h[hd][b:b + 1, :]
                                 for hd in range(NUM_HEADS)], axis=0)
        bet_b = jnp.concatenate([bet_h[hd][b:b + 1, :]
                                 for hd in range(NUM_HEADS)], axis=0)
        g_b = jnp.concatenate([g_h[hd][b:b + 1, :]
                               for hd in range(NUM_HEADS)], axis=0)
        gam_b = jnp.concatenate([gam_h[hd][b:b + 1, :]
                                 for hd in range(NUM_HEADS)], axis=0)
        sh_b = jnp.concatenate([sh_h[hd][b:b + 1, :]
                                for hd in range(NUM_HEADS)], axis=0)

        dot_b = _dot(key_b, mem_b)
        mn2 = _dot(ones_row, mem_b * mem_b)
        kn2 = jnp.sum(key_b * key_b, axis=-1, keepdims=True)
        denom = jnp.sqrt(kn2) * jnp.sqrt(mn2) + EPS
        wc = bet_b * dot_b / denom
        wc = wc - jnp.max(wc, axis=-1, keepdims=True)
        wc = jnp.exp(wc)
        wc = wc / jnp.sum(wc, axis=-1, keepdims=True)

        wmix_b = wmix_ref[b]
        wg = g_b * wc + (1.0 - g_b) * wmix_b

        roll_m = jnp.concatenate([wg[:, 1:], wg[:, 0:1]], axis=-1)
        roll_p = jnp.concatenate([wg[:, -1:], wg[:, :-1]], axis=-1)
        ws = (roll_m * sh_b[:, 0:1] + wg * sh_b[:, 1:2]
              + roll_p * sh_b[:, 2:3])

        wp = jnp.exp2(gam_b * jnp.log2(ws + 1e-12))
        w_b = wp / jnp.sum(wp, axis=-1, keepdims=True)
        wtn_ref[b] = w_b

        er_b = jnp.concatenate([erase_h[hd][b:b + 1, :]
                                for hd in range(NUM_HEADS)], axis=0)
        w2 = w_b * w_b
        wpow = (w_b, w2, w2 * w_b, w2 * w2)
        e2f = er_b * er_b
        epow = (er_b, e2f, e2f * er_b, e2f * e2f)
        NC = N_ADDR // 1024
        for c in range(NC):
            sl = slice(c * 1024, (c + 1) * 1024)
            pT = [_dot_t_lhs(epow[k], wpow[k][:, sl]) for k in range(4)]
            p1, p2, p3, p4 = pT
            el2 = 0.5 * (p1 * p1 - p2)
            el3 = (el2 * p1 - p1 * p2 + p3) * (1.0 / 3.0)
            el4 = (el3 * p1 - el2 * p2 + p1 * p3 - p4) * 0.25
            keep = (1.0 - p1) + (el2 - el3) + el4
            added_c = _dot_t_lhs(add_b, w_b[:, sl])
            mem_buf[slot, b, :, sl] = mem_b[:, sl] * keep + added_c

    out_copy(i, slot).start()

    @pl.when(i == GRID - 1)
    def _():
        out_copy(i, slot).wait()


def kernel(tm_input, tm_state, wt, mem, wt_address_dynamic,
           W_f, b_f, W_h, b_h, W_s, b_s, W_o, b_o, W_u, b_u):
    grid = (B // BB,)

    def bmap(i):
        return (i, 0)

    def bmap3(i):
        return (i, 0, 0)

    def const2(i):
        return (0, 0)

    f32 = jnp.float32
    out_shapes = (
        jax.ShapeDtypeStruct((B, OUT_DIM), f32),
        jax.ShapeDtypeStruct((B, STATE), f32),
        jax.ShapeDtypeStruct((B, NUM_HEADS, N_ADDR), f32),
        jax.ShapeDtypeStruct((B, M, N_ADDR), f32),
        jax.ShapeDtypeStruct((B, NUM_HEADS, N_ADDR), f32),
    )
    in_specs = [
        pl.BlockSpec((BB, IN_DIM), bmap),
        pl.BlockSpec((BB, STATE), bmap),
        pl.BlockSpec((BB, NUM_HEADS, N_ADDR), bmap3),
        pl.BlockSpec(memory_space=pl.ANY),
        pl.BlockSpec((BB, NUM_HEADS, N_ADDR), bmap3),
        pl.BlockSpec((STATE + IN_DIM, NUM_HEADS), const2),
        pl.BlockSpec((1, NUM_HEADS), const2),
        pl.BlockSpec((STATE + IN_DIM, 2), const2),
        pl.BlockSpec((1, 2), const2),
        pl.BlockSpec((CTRL_IN, STATE), const2),
        pl.BlockSpec((1, STATE), const2),
        pl.BlockSpec((CTRL_IN, OUT_DIM), const2),
        pl.BlockSpec((1, OUT_DIM), const2),
        pl.BlockSpec((CTRL_IN, UPDATE_SIZE), const2),
        pl.BlockSpec((1, UPDATE_SIZE), const2),
    ]
    out_specs = (
        pl.BlockSpec((BB, OUT_DIM), bmap),
        pl.BlockSpec((BB, STATE), bmap),
        pl.BlockSpec((BB, NUM_HEADS, N_ADDR), bmap3),
        pl.BlockSpec(memory_space=pl.ANY),
        pl.BlockSpec((BB, NUM_HEADS, N_ADDR), bmap3),
    )
    out, sn, wtn, memn_t, wtdyn = pl.pallas_call(
        _ntm_kernel,
        grid=grid,
        in_specs=in_specs,
        out_specs=out_specs,
        out_shape=out_shapes,
        scratch_shapes=[
            pltpu.VMEM((BB, NUM_HEADS, N_ADDR), f32),
            pltpu.VMEM((2, BB, M, N_ADDR), f32),
            pltpu.SemaphoreType.DMA((2,)),
            pltpu.SemaphoreType.DMA((2,)),
        ],
        compiler_params=pltpu.CompilerParams(
            dimension_semantics=("arbitrary",),
            vmem_limit_bytes=60 * 1024 * 1024,
        ),
        name="ntm_cell_fused",
    )(tm_input, tm_state, wt, jnp.transpose(mem, (0, 2, 1)), wt_address_dynamic,
      W_f.astype(jnp.bfloat16), b_f.reshape(1, -1),
      W_h.astype(jnp.bfloat16), b_h.reshape(1, -1),
      W_s.astype(jnp.bfloat16), b_s.reshape(1, -1),
      W_o.astype(jnp.bfloat16), b_o.reshape(1, -1),
      W_u.astype(jnp.bfloat16), b_u.reshape(1, -1))
    return out, sn, wtn, jnp.transpose(memn_t, (0, 2, 1)), wtdyn


# speedup vs baseline: 1.0484x; 1.0484x over previous
import jax
import jax.numpy as jnp
from jax import lax
from jax.experimental import pallas as pl
from jax.experimental.pallas import tpu as pltpu

NUM_HEADS = 4
M = 64
NUM_SHIFT = 3
B, IN_DIM, STATE, OUT_DIM, N_ADDR = 128, 256, 512, 256, 2048
EPS = 1e-8
PER_HEAD = 2 * M + NUM_SHIFT + 1 + (M + 2)
UPDATE_SIZE = NUM_HEADS * PER_HEAD
READ_SIZE = NUM_HEADS * M
CTRL_IN = IN_DIM + STATE + READ_SIZE

BB = 16
GRID = B // BB


def _sigmoid(x):
    return 1.0 / (1.0 + jnp.exp(-x))


def _softplus(x):
    return jnp.maximum(x, 0.0) + jnp.log(1.0 + jnp.exp(-jnp.abs(x)))


def _dot(a, b):
    return jnp.dot(a, b, preferred_element_type=jnp.float32)


def _dot_t_rhs(a, b):
    return lax.dot_general(a, b, (((1,), (1,)), ((), ())),
                           preferred_element_type=jnp.float32)


def _dot_t_lhs(a, b):
    return lax.dot_general(a, b, (((0,), (0,)), ((), ())),
                           preferred_element_type=jnp.float32)


def _ntm_kernel(x_ref, s_ref, wt_ref, mem_hbm, wtd_ref,
                W_f_ref, b_f_ref, W_h_ref, b_h_ref,
                W_s_ref, b_s_ref, W_o_ref, b_o_ref, W_u_ref, b_u_ref,
                out_ref, sn_ref, wtn_ref, memn_hbm, wtdyn_ref,
                wmix_ref, mem_buf, in_sems, out_sems):
    i = pl.program_id(0)

    def in_copy(step, slot):
        return pltpu.make_async_copy(
            mem_hbm.at[pl.ds(step * BB, BB)],
            mem_buf.at[slot],
            in_sems.at[slot],
        )

    def out_copy(step, slot):
        return pltpu.make_async_copy(
            mem_buf.at[slot],
            memn_hbm.at[pl.ds(step * BB, BB)],
            out_sems.at[slot],
        )

    @pl.when(i == 0)
    def _():
        in_copy(0, 0).start()

    @pl.when(i >= 1)
    def _():
        out_copy(i - 1, (i + 1) % 2).wait()

    @pl.when(i + 1 < GRID)
    def _():
        in_copy(i + 1, (i + 1) % 2).start()

    slot = i % 2
    in_copy(i, slot).wait()
    mem_ref = mem_buf.at[slot]

    x = x_ref[...].astype(jnp.bfloat16)
    s = s_ref[...].astype(jnp.bfloat16)

    f = _sigmoid(_dot(s, W_f_ref[0:STATE, :]) + _dot(x, W_f_ref[STATE:, :])
                 + b_f_ref[...])
    h = _sigmoid(_dot(s, W_h_ref[0:STATE, :]) + _dot(x, W_h_ref[STATE:, :])
                 + b_h_ref[...])

    hh = lax.broadcasted_iota(jnp.int32, (NUM_HEADS, N_ADDR), 0)
    nn = lax.broadcasted_iota(jnp.int32, (NUM_HEADS, N_ADDR), 1)
    delta = jnp.where((hh == 0) & (nn == 0), 1.0, 0.0)

    reads = []
    for b in range(BB):
        wt_b = wt_ref[b]
        wtd_b = wtd_ref[b]
        f0 = f[b:b + 1, 0:1]
        f1 = f[b:b + 1, 1:2]
        h0 = h[b:b + 1, 0:1]
        wt_dyn_b = (1.0 - h0) * wt_b + h0 * wtd_b
        wtdyn_ref[b] = wt_dyn_b
        wmix_b = ((1.0 - f0) * (1.0 - f1) * delta
                  + (1.0 - f0) * f1 * wt_b
                  + f0 * (1.0 - f1) * wt_dyn_b)
        wmix_ref[b] = wmix_b
        reads.append(_dot_t_rhs(wmix_b, mem_ref[b]))

    read_h = [jnp.concatenate([reads[b][hd:hd + 1, :] for b in range(BB)],
                              axis=0) for hd in range(NUM_HEADS)]

    read_bf = [r.astype(jnp.bfloat16) for r in read_h]

    def ctrl_mm(W_ref, bias_ref):
        z = (_dot(x, W_ref[0:IN_DIM, :])
             + _dot(s, W_ref[IN_DIM:IN_DIM + STATE, :]))
        for hd in range(NUM_HEADS):
            r0 = IN_DIM + STATE + hd * M
            z = z + _dot(read_bf[hd], W_ref[r0:r0 + M, :])
        return z + bias_ref[...]

    sn_ref[...] = _sigmoid(ctrl_mm(W_s_ref, b_s_ref))
    out_ref[...] = _sigmoid(ctrl_mm(W_o_ref, b_o_ref))
    u = ctrl_mm(W_u_ref, b_u_ref)

    erase_h, add_h, key_h, sh_h, gam_h, bet_h, g_h = [], [], [], [], [], [], []
    for hd in range(NUM_HEADS):
        o = hd * PER_HEAD
        erase_h.append(_sigmoid(u[:, o:o + M]))
        add_h.append(u[:, o + M:o + 2 * M])
        raw_sh = u[:, o + 2 * M:o + 2 * M + NUM_SHIFT]
        raw_sh = raw_sh - jnp.max(raw_sh, axis=-1, keepdims=True)
        e = jnp.exp(raw_sh)
        sh_h.append(e / jnp.sum(e, axis=-1, keepdims=True))
        gam_h.append(1.0 + _softplus(
            u[:, o + 2 * M + NUM_SHIFT:o + 2 * M + NUM_SHIFT + 1]))
        co = o + 2 * M + NUM_SHIFT + 1
        key_h.append(u[:, co:co + M])
        bet_h.append(_softplus(u[:, co + M:co + M + 1]))
        g_h.append(_sigmoid(u[:, co + M + 1:co + M + 2]))

    ones_row = jnp.full((1, M), 1.0, dtype=jnp.float32)

    for b in range(BB):
        mem_b = mem_ref[b]
        key_b = jnp.concatenate([key_h[hd][b:b + 1, :]
                                 for hd in range(NUM_HEADS)], axis=0)
        add_b = jnp.concatenate([add_h[hd][b:b + 1, :]
                                 for hd in range(NUM_HEADS)], axis=0)
        bet_b = jnp.concatenate([bet_h[hd][b:b + 1, :]
                                 for hd in range(NUM_HEADS)], axis=0)
        g_b = jnp.concatenate([g_h[hd][b:b + 1, :]
                               for hd in range(NUM_HEADS)], axis=0)
        gam_b = jnp.concatenate([gam_h[hd][b:b + 1, :]
                                 for hd in range(NUM_HEADS)], axis=0)
        sh_b = jnp.concatenate([sh_h[hd][b:b + 1, :]
                                for hd in range(NUM_HEADS)], axis=0)

        dot_b = _dot(key_b, mem_b)
        mn2 = _dot(ones_row, mem_b * mem_b)
        kn2 = jnp.sum(key_b * key_b, axis=-1, keepdims=True)
        denom = jnp.sqrt(kn2) * jnp.sqrt(mn2) + EPS
        wc = bet_b * dot_b / denom
        wc = wc - jnp.max(wc, axis=-1, keepdims=True)
        wc = jnp.exp(wc)
        wc = wc / jnp.sum(wc, axis=-1, keepdims=True)

        wmix_b = wmix_ref[b]
        wg = g_b * wc + (1.0 - g_b) * wmix_b

        roll_m = jnp.concatenate([wg[:, 1:], wg[:, 0:1]], axis=-1)
        roll_p = jnp.concatenate([wg[:, -1:], wg[:, :-1]], axis=-1)
        ws = (roll_m * sh_b[:, 0:1] + wg * sh_b[:, 1:2]
              + roll_p * sh_b[:, 2:3])

        wp = jnp.exp2(gam_b * jnp.log2(ws + 1e-12))
        w_b = wp / jnp.sum(wp, axis=-1, keepdims=True)
        wtn_ref[b] = w_b

        er_b = jnp.concatenate([erase_h[hd][b:b + 1, :]
                                for hd in range(NUM_HEADS)], axis=0)
        w2 = w_b * w_b
        wpow = (w_b, w2, w2 * w_b, w2 * w2)
        e2f = er_b * er_b
        epow = (er_b, e2f, e2f * er_b, e2f * e2f)
        NC = N_ADDR // 1024
        for c in range(NC):
            sl = slice(c * 1024, (c + 1) * 1024)
            pT = [_dot_t_lhs(epow[k], wpow[k][:, sl]) for k in range(4)]
            p1, p2, p3, p4 = pT
            el2 = 0.5 * (p1 * p1 - p2)
            el3 = (el2 * p1 - p1 * p2 + p3) * (1.0 / 3.0)
            el4 = (el3 * p1 - el2 * p2 + p1 * p3 - p4) * 0.25
            keep = (1.0 - p1) + (el2 - el3) + el4
            added_c = _dot_t_lhs(add_b, w_b[:, sl])
            mem_buf[slot, b, :, sl] = mem_b[:, sl] * keep + added_c

    out_copy(i, slot).start()

    @pl.when(i == GRID - 1)
    def _():
        out_copy(i, slot).wait()


def kernel(tm_input, tm_state, wt, mem, wt_address_dynamic,
           W_f, b_f, W_h, b_h, W_s, b_s, W_o, b_o, W_u, b_u):
    grid = (B // BB,)

    def bmap(i):
        return (i, 0)

    def bmap3(i):
        return (i, 0, 0)

    def const2(i):
        return (0, 0)

    f32 = jnp.float32
    out_shapes = (
        jax.ShapeDtypeStruct((B, OUT_DIM), f32),
        jax.ShapeDtypeStruct((B, STATE), f32),
        jax.ShapeDtypeStruct((B, NUM_HEADS, N_ADDR), f32),
        jax.ShapeDtypeStruct((B, M, N_ADDR), f32),
        jax.ShapeDtypeStruct((B, NUM_HEADS, N_ADDR), f32),
    )
    in_specs = [
        pl.BlockSpec((BB, IN_DIM), bmap),
        pl.BlockSpec((BB, STATE), bmap),
        pl.BlockSpec((BB, NUM_HEADS, N_ADDR), bmap3),
        pl.BlockSpec(memory_space=pl.ANY),
        pl.BlockSpec((BB, NUM_HEADS, N_ADDR), bmap3),
        pl.BlockSpec((STATE + IN_DIM, NUM_HEADS), const2),
        pl.BlockSpec((1, NUM_HEADS), const2),
        pl.BlockSpec((STATE + IN_DIM, 2), const2),
        pl.BlockSpec((1, 2), const2),
        pl.BlockSpec((CTRL_IN, STATE), const2),
        pl.BlockSpec((1, STATE), const2),
        pl.BlockSpec((CTRL_IN, OUT_DIM), const2),
        pl.BlockSpec((1, OUT_DIM), const2),
        pl.BlockSpec((CTRL_IN, UPDATE_SIZE), const2),
        pl.BlockSpec((1, UPDATE_SIZE), const2),
    ]
    out_specs = (
        pl.BlockSpec((BB, OUT_DIM), bmap),
        pl.BlockSpec((BB, STATE), bmap),
        pl.BlockSpec((BB, NUM_HEADS, N_ADDR), bmap3),
        pl.BlockSpec(memory_space=pl.ANY),
        pl.BlockSpec((BB, NUM_HEADS, N_ADDR), bmap3),
    )
    out, sn, wtn, memn_t, wtdyn = pl.pallas_call(
        _ntm_kernel,
        grid=grid,
        in_specs=in_specs,
        out_specs=out_specs,
        out_shape=out_shapes,
        scratch_shapes=[
            pltpu.VMEM((BB, NUM_HEADS, N_ADDR), f32),
            pltpu.VMEM((2, BB, M, N_ADDR), f32),
            pltpu.SemaphoreType.DMA((2,)),
            pltpu.SemaphoreType.DMA((2,)),
        ],
        compiler_params=pltpu.CompilerParams(
            dimension_semantics=("arbitrary",),
            vmem_limit_bytes=60 * 1024 * 1024,
        ),
        name="ntm_cell_fused",
    )(tm_input, tm_state, wt, jnp.transpose(mem, (0, 2, 1)), wt_address_dynamic,
      W_f.astype(jnp.bfloat16), b_f.reshape(1, -1),
      W_h.astype(jnp.bfloat16), b_h.reshape(1, -1),
      W_s.astype(jnp.bfloat16), b_s.reshape(1, -1),
      W_o.astype(jnp.bfloat16), b_o.reshape(1, -1),
      W_u.astype(jnp.bfloat16), b_u.reshape(1, -1))
    return out, sn, wtn, jnp.transpose(memn_t, (0, 2, 1)), wtdyn


# speedup vs baseline: 1.0654x; 1.0163x over previous
import jax
import jax.numpy as jnp
from jax import lax
from jax.experimental import pallas as pl
from jax.experimental.pallas import tpu as pltpu

NUM_HEADS = 4
M = 64
NUM_SHIFT = 3
B, IN_DIM, STATE, OUT_DIM, N_ADDR = 128, 256, 512, 256, 2048
EPS = 1e-8
PER_HEAD = 2 * M + NUM_SHIFT + 1 + (M + 2)
UPDATE_SIZE = NUM_HEADS * PER_HEAD
READ_SIZE = NUM_HEADS * M
CTRL_IN = IN_DIM + STATE + READ_SIZE

BB = 32
GRID = B // BB


def _sigmoid(x):
    return 1.0 / (1.0 + jnp.exp(-x))


def _softplus(x):
    return jnp.maximum(x, 0.0) + jnp.log(1.0 + jnp.exp(-jnp.abs(x)))


def _dot(a, b):
    return jnp.dot(a, b, preferred_element_type=jnp.float32)


def _dot_t_rhs(a, b):
    return lax.dot_general(a, b, (((1,), (1,)), ((), ())),
                           preferred_element_type=jnp.float32)


def _dot_t_lhs(a, b):
    return lax.dot_general(a, b, (((0,), (0,)), ((), ())),
                           preferred_element_type=jnp.float32)


def _ntm_kernel(x_ref, s_ref, wt_ref, mem_hbm, wtd_ref,
                W_f_ref, b_f_ref, W_h_ref, b_h_ref,
                W_s_ref, b_s_ref, W_o_ref, b_o_ref, W_u_ref, b_u_ref,
                out_ref, sn_ref, wtn_ref, memn_hbm, wtdyn_ref,
                wmix_ref, mem_buf, in_sems, out_sems):
    i = pl.program_id(0)

    def in_copy(step, slot):
        return pltpu.make_async_copy(
            mem_hbm.at[pl.ds(step * BB, BB)],
            mem_buf.at[slot],
            in_sems.at[slot],
        )

    def out_copy(step, slot):
        return pltpu.make_async_copy(
            mem_buf.at[slot],
            memn_hbm.at[pl.ds(step * BB, BB)],
            out_sems.at[slot],
        )

    @pl.when(i == 0)
    def _():
        in_copy(0, 0).start()

    @pl.when(i >= 1)
    def _():
        out_copy(i - 1, (i + 1) % 2).wait()

    @pl.when(i + 1 < GRID)
    def _():
        in_copy(i + 1, (i + 1) % 2).start()

    slot = i % 2
    in_copy(i, slot).wait()
    mem_ref = mem_buf.at[slot]

    x = x_ref[...].astype(jnp.bfloat16)
    s = s_ref[...].astype(jnp.bfloat16)

    f = _sigmoid(_dot(s, W_f_ref[0:STATE, :]) + _dot(x, W_f_ref[STATE:, :])
                 + b_f_ref[...])
    h = _sigmoid(_dot(s, W_h_ref[0:STATE, :]) + _dot(x, W_h_ref[STATE:, :])
                 + b_h_ref[...])

    hh = lax.broadcasted_iota(jnp.int32, (NUM_HEADS, N_ADDR), 0)
    nn = lax.broadcasted_iota(jnp.int32, (NUM_HEADS, N_ADDR), 1)
    delta = jnp.where((hh == 0) & (nn == 0), 1.0, 0.0)

    reads = []
    for b in range(BB):
        wt_b = wt_ref[b]
        wtd_b = wtd_ref[b]
        f0 = f[b:b + 1, 0:1]
        f1 = f[b:b + 1, 1:2]
        h0 = h[b:b + 1, 0:1]
        wt_dyn_b = (1.0 - h0) * wt_b + h0 * wtd_b
        wtdyn_ref[b] = wt_dyn_b
        wmix_b = ((1.0 - f0) * (1.0 - f1) * delta
                  + (1.0 - f0) * f1 * wt_b
                  + f0 * (1.0 - f1) * wt_dyn_b)
        wmix_ref[b] = wmix_b
        reads.append(_dot_t_rhs(wmix_b, mem_ref[b]))

    read_h = [jnp.concatenate([reads[b][hd:hd + 1, :] for b in range(BB)],
                              axis=0) for hd in range(NUM_HEADS)]

    read_bf = [r.astype(jnp.bfloat16) for r in read_h]

    def ctrl_mm(W_ref, bias_ref):
        z = (_dot(x, W_ref[0:IN_DIM, :])
             + _dot(s, W_ref[IN_DIM:IN_DIM + STATE, :]))
        for hd in range(NUM_HEADS):
            r0 = IN_DIM + STATE + hd * M
            z = z + _dot(read_bf[hd], W_ref[r0:r0 + M, :])
        return z + bias_ref[...]

    sn_ref[...] = _sigmoid(ctrl_mm(W_s_ref, b_s_ref))
    out_ref[...] = _sigmoid(ctrl_mm(W_o_ref, b_o_ref))
    u = ctrl_mm(W_u_ref, b_u_ref)

    erase_h, add_h, key_h, sh_h, gam_h, bet_h, g_h = [], [], [], [], [], [], []
    for hd in range(NUM_HEADS):
        o = hd * PER_HEAD
        erase_h.append(_sigmoid(u[:, o:o + M]))
        add_h.append(u[:, o + M:o + 2 * M])
        raw_sh = u[:, o + 2 * M:o + 2 * M + NUM_SHIFT]
        raw_sh = raw_sh - jnp.max(raw_sh, axis=-1, keepdims=True)
        e = jnp.exp(raw_sh)
        sh_h.append(e / jnp.sum(e, axis=-1, keepdims=True))
        gam_h.append(1.0 + _softplus(
            u[:, o + 2 * M + NUM_SHIFT:o + 2 * M + NUM_SHIFT + 1]))
        co = o + 2 * M + NUM_SHIFT + 1
        key_h.append(u[:, co:co + M])
        bet_h.append(_softplus(u[:, co + M:co + M + 1]))
        g_h.append(_sigmoid(u[:, co + M + 1:co + M + 2]))

    ones_row = jnp.full((1, M), 1.0, dtype=jnp.float32)

    for b in range(BB):
        mem_b = mem_ref[b]
        key_b = jnp.concatenate([key_h[hd][b:b + 1, :]
                                 for hd in range(NUM_HEADS)], axis=0)
        add_b = jnp.concatenate([add_h[hd][b:b + 1, :]
                                 for hd in range(NUM_HEADS)], axis=0)
        bet_b = jnp.concatenate([bet_h[hd][b:b + 1, :]
                                 for hd in range(NUM_HEADS)], axis=0)
        g_b = jnp.concatenate([g_h[hd][b:b + 1, :]
                               for hd in range(NUM_HEADS)], axis=0)
        gam_b = jnp.concatenate([gam_h[hd][b:b + 1, :]
                                 for hd in range(NUM_HEADS)], axis=0)
        sh_b = jnp.concatenate([sh_h[hd][b:b + 1, :]
                                for hd in range(NUM_HEADS)], axis=0)

        dot_b = _dot(key_b, mem_b)
        mn2 = _dot(ones_row, mem_b * mem_b)
        kn2 = jnp.sum(key_b * key_b, axis=-1, keepdims=True)
        denom = jnp.sqrt(kn2) * jnp.sqrt(mn2) + EPS
        wc = bet_b * dot_b / denom
        wc = wc - jnp.max(wc, axis=-1, keepdims=True)
        wc = jnp.exp(wc)
        wc = wc / jnp.sum(wc, axis=-1, keepdims=True)

        wmix_b = wmix_ref[b]
        wg = g_b * wc + (1.0 - g_b) * wmix_b

        roll_m = jnp.concatenate([wg[:, 1:], wg[:, 0:1]], axis=-1)
        roll_p = jnp.concatenate([wg[:, -1:], wg[:, :-1]], axis=-1)
        ws = (roll_m * sh_b[:, 0:1] + wg * sh_b[:, 1:2]
              + roll_p * sh_b[:, 2:3])

        wp = jnp.exp2(gam_b * jnp.log2(ws + 1e-12))
        w_b = wp / jnp.sum(wp, axis=-1, keepdims=True)
        wtn_ref[b] = w_b

        er_b = jnp.concatenate([erase_h[hd][b:b + 1, :]
                                for hd in range(NUM_HEADS)], axis=0)
        w2 = w_b * w_b
        wpow = (w_b, w2, w2 * w_b, w2 * w2)
        e2f = er_b * er_b
        epow = (er_b, e2f, e2f * er_b, e2f * e2f)
        NC = N_ADDR // 1024
        for c in range(NC):
            sl = slice(c * 1024, (c + 1) * 1024)
            pT = [_dot_t_lhs(epow[k], wpow[k][:, sl]) for k in range(4)]
            p1, p2, p3, p4 = pT
            el2 = 0.5 * (p1 * p1 - p2)
            el3 = (el2 * p1 - p1 * p2 + p3) * (1.0 / 3.0)
            el4 = (el3 * p1 - el2 * p2 + p1 * p3 - p4) * 0.25
            keep = (1.0 - p1) + (el2 - el3) + el4
            added_c = _dot_t_lhs(add_b, w_b[:, sl])
            mem_buf[slot, b, :, sl] = mem_b[:, sl] * keep + added_c

    out_copy(i, slot).start()

    @pl.when(i == GRID - 1)
    def _():
        out_copy(i, slot).wait()


def kernel(tm_input, tm_state, wt, mem, wt_address_dynamic,
           W_f, b_f, W_h, b_h, W_s, b_s, W_o, b_o, W_u, b_u):
    grid = (B // BB,)

    def bmap(i):
        return (i, 0)

    def bmap3(i):
        return (i, 0, 0)

    def const2(i):
        return (0, 0)

    f32 = jnp.float32
    out_shapes = (
        jax.ShapeDtypeStruct((B, OUT_DIM), f32),
        jax.ShapeDtypeStruct((B, STATE), f32),
        jax.ShapeDtypeStruct((B, NUM_HEADS, N_ADDR), f32),
        jax.ShapeDtypeStruct((B, M, N_ADDR), f32),
        jax.ShapeDtypeStruct((B, NUM_HEADS, N_ADDR), f32),
    )
    in_specs = [
        pl.BlockSpec((BB, IN_DIM), bmap),
        pl.BlockSpec((BB, STATE), bmap),
        pl.BlockSpec((BB, NUM_HEADS, N_ADDR), bmap3),
        pl.BlockSpec(memory_space=pl.ANY),
        pl.BlockSpec((BB, NUM_HEADS, N_ADDR), bmap3),
        pl.BlockSpec((STATE + IN_DIM, NUM_HEADS), const2),
        pl.BlockSpec((1, NUM_HEADS), const2),
        pl.BlockSpec((STATE + IN_DIM, 2), const2),
        pl.BlockSpec((1, 2), const2),
        pl.BlockSpec((CTRL_IN, STATE), const2),
        pl.BlockSpec((1, STATE), const2),
        pl.BlockSpec((CTRL_IN, OUT_DIM), const2),
        pl.BlockSpec((1, OUT_DIM), const2),
        pl.BlockSpec((CTRL_IN, UPDATE_SIZE), const2),
        pl.BlockSpec((1, UPDATE_SIZE), const2),
    ]
    out_specs = (
        pl.BlockSpec((BB, OUT_DIM), bmap),
        pl.BlockSpec((BB, STATE), bmap),
        pl.BlockSpec((BB, NUM_HEADS, N_ADDR), bmap3),
        pl.BlockSpec(memory_space=pl.ANY),
        pl.BlockSpec((BB, NUM_HEADS, N_ADDR), bmap3),
    )
    out, sn, wtn, memn_t, wtdyn = pl.pallas_call(
        _ntm_kernel,
        grid=grid,
        in_specs=in_specs,
        out_specs=out_specs,
        out_shape=out_shapes,
        scratch_shapes=[
            pltpu.VMEM((BB, NUM_HEADS, N_ADDR), f32),
            pltpu.VMEM((2, BB, M, N_ADDR), f32),
            pltpu.SemaphoreType.DMA((2,)),
            pltpu.SemaphoreType.DMA((2,)),
        ],
        compiler_params=pltpu.CompilerParams(
            dimension_semantics=("arbitrary",),
            vmem_limit_bytes=60 * 1024 * 1024,
        ),
        name="ntm_cell_fused",
    )(tm_input, tm_state, wt, jnp.transpose(mem, (0, 2, 1)), wt_address_dynamic,
      W_f.astype(jnp.bfloat16), b_f.reshape(1, -1),
      W_h.astype(jnp.bfloat16), b_h.reshape(1, -1),
      W_s.astype(jnp.bfloat16), b_s.reshape(1, -1),
      W_o.astype(jnp.bfloat16), b_o.reshape(1, -1),
      W_u.astype(jnp.bfloat16), b_u.reshape(1, -1))
    return out, sn, wtn, jnp.transpose(memn_t, (0, 2, 1)), wtdyn
